# baseline (speedup 1.0000x reference)
import functools

import jax
import jax.numpy as jnp
from jax import lax
from jax.experimental import pallas as pl
from jax.experimental.pallas import tpu as pltpu
from jax.experimental.pallas import tpu_sc as plsc

_N = 16384
_EMB = 64
_PAIR = 128
_VBLK = 65536
_Q = _VBLK // 4
_ENT_V = 1000000
_ENT_GRID = -(-_ENT_V // _VBLK)
_ENT_ROWS = _ENT_GRID * _Q
_REL_ROWS = _Q
_NC = 2
_NS = 16
_NW = _NC * _NS
_BPW = _N // _NW
_CHUNK = 128
_L = 16
_SH = _VBLK.bit_length() - 1
_QSH = _SH - 2
_QMASK = _Q - 1


def _pack16(lo, hi):
  lo16 = lax.bitcast_convert_type(lo.astype(jnp.bfloat16), jnp.uint16)
  hi16 = lax.bitcast_convert_type(hi.astype(jnp.bfloat16), jnp.uint16)
  u = lo16.astype(jnp.uint32) | (hi16.astype(jnp.uint32) << 16)
  return lax.bitcast_convert_type(u, jnp.float32)


def _proj_body(xt_ref, w_ref, b_ref, out_ref):
  xt = xt_ref[...]
  w = w_ref[...]
  b = b_ref[...]
  dn = (((0,), (0,)), ((), ()))
  ys = []
  for t in range(4):
    y = lax.dot_general(xt[:, t * _Q:(t + 1) * _Q], w, dn,
                        preferred_element_type=jnp.float32)
    ys.append(y + b)
  out_ref[:, :_EMB] = _pack16(ys[0], ys[1])
  out_ref[:, _EMB:] = _pack16(ys[2], ys[3])


def _project(table, W, b, grid, out_rows):
  tt = table.T
  return pl.pallas_call(
      _proj_body,
      grid=(grid,),
      in_specs=[
          pl.BlockSpec((_EMB, _VBLK), lambda j: (0, j)),
          pl.BlockSpec((_EMB, _EMB), lambda j: (0, 0)),
          pl.BlockSpec((1, _EMB), lambda j: (0, 0)),
      ],
      out_specs=pl.BlockSpec((_Q, _PAIR), lambda j: (j, 0)),
      out_shape=jax.ShapeDtypeStruct((out_rows, _PAIR), jnp.float32),
  )(tt, W, b.reshape(1, _EMB))


def _sc_gather(s, r, o, p4_ent, p4_rel):
  mesh = plsc.VectorSubcoreMesh(
      core_axis_name="c", subcore_axis_name="s",
      num_cores=_NC, num_subcores=_NS)

  @functools.partial(
      pl.kernel,
      out_type=[jax.ShapeDtypeStruct((_N, _PAIR), jnp.float32)] * 3,
      mesh=mesh,
      scratch_types=[
          pltpu.VMEM((_BPW,), jnp.int32),
          pltpu.VMEM((_BPW,), jnp.int32),
          pltpu.VMEM((_BPW,), jnp.int32),
          pltpu.VMEM((_BPW // 2, _PAIR), jnp.float32),
          pltpu.VMEM((_BPW // 2, _PAIR), jnp.float32),
          pltpu.VMEM((_BPW // 2, _PAIR), jnp.float32),
          pltpu.SemaphoreType.DMA,
          pltpu.SemaphoreType.DMA,
      ],
  )
  def k(s_h, r_h, o_h, ent_h, rel_h, xs_h, xr_h, xo_h,
        si_v, ri_v, oi_v, gs_v, gr_v, go_v, gsem, wsem):
    wid = lax.axis_index("s") * _NC + lax.axis_index("c")
    base = wid * _BPW
    pltpu.sync_copy(s_h.at[pl.ds(base, _BPW)], si_v)
    pltpu.sync_copy(r_h.at[pl.ds(base, _BPW)], ri_v)
    pltpu.sync_copy(o_h.at[pl.ds(base, _BPW)], oi_v)
    for iv in (si_v, ri_v, oi_v):
      for g in range(_BPW // _L):
        sl = pl.ds(g * _L, _L)
        v = iv[sl]
        iv[sl] = ((v >> _SH) << _QSH) | (v & _QMASK)
    hr = _BPW // 2
    prev_wb = []
    for h in range(2):
      for c in prev_wb:
        c.wait()
      copies = []
      for j in range(hr // _CHUNK):
        isl = pl.ds(h * hr + j * _CHUNK, _CHUNK)
        bsl = pl.ds(j * _CHUNK, _CHUNK)
        copies.append(
            pltpu.async_copy(ent_h.at[si_v.at[isl]], gs_v.at[bsl], gsem))
        copies.append(
            pltpu.async_copy(rel_h.at[ri_v.at[isl]], gr_v.at[bsl], gsem))
        copies.append(
            pltpu.async_copy(ent_h.at[oi_v.at[isl]], go_v.at[bsl], gsem))
      for c in copies:
        c.wait()
      osl = pl.ds(base + h * hr, hr)
      prev_wb = [pltpu.async_copy(gs_v, xs_h.at[osl], wsem),
                 pltpu.async_copy(gr_v, xr_h.at[osl], wsem),
                 pltpu.async_copy(go_v, xo_h.at[osl], wsem)]
    for c in prev_wb:
      c.wait()

  return k(s, r, o, p4_ent, p4_rel)


_BLK = 2048


def _sel_body(xs_ref, xr_ref, xo_ref, s_ref, r_ref, o_ref, out_ref):
  for col, x_ref, i_ref in ((0, xs_ref, s_ref), (1, xr_ref, r_ref),
                            (2, xo_ref, o_ref)):
    x2 = x_ref[...]
    idx = i_ref[...]
    grp = (idx >> (_QSH + 1)) & 1
    hi = (idx >> _QSH) & 1
    xh = jnp.where(grp > 0, x2[:, _EMB:], x2[:, :_EMB])
    u = lax.bitcast_convert_type(xh, jnp.uint32)
    picked = jnp.where(hi > 0, u & jnp.uint32(0xFFFF0000), u << 16)
    sel = lax.bitcast_convert_type(picked, jnp.float32)
    out_ref[col * _EMB:(col + 1) * _EMB, :] = sel.T


def _select_concat(xs2, xr2, xo2, s, r, o):
  row_spec = pl.BlockSpec((_BLK, _PAIR), lambda i: (i, 0))
  i_spec = pl.BlockSpec((_BLK, 1), lambda i: (i, 0))
  return pl.pallas_call(
      _sel_body,
      grid=(_N // _BLK,),
      in_specs=[row_spec, row_spec, row_spec, i_spec, i_spec, i_spec],
      out_specs=pl.BlockSpec((3 * _EMB, _BLK), lambda i: (0, i)),
      out_shape=jax.ShapeDtypeStruct((3 * _EMB, _N), jnp.float32),
  )(xs2, xr2, xo2, s.reshape(_N, 1), r.reshape(_N, 1), o.reshape(_N, 1)).T


def kernel(s, r, o, entity_table, relation_table, W, b):
  s = s.astype(jnp.int32)
  r = r.astype(jnp.int32)
  o = o.astype(jnp.int32)
  p4_ent = _project(entity_table, W, b, _ENT_GRID, _ENT_ROWS)
  p4_rel = _project(relation_table, W, b, 1, _REL_ROWS)
  xs2, xr2, xo2 = _sc_gather(s, r, o, p4_ent, p4_rel)
  return _select_concat(xs2, xr2, xo2, s, r, o)


# speedup vs baseline: 1.0477x; 1.0477x over previous
import functools

import jax
import jax.numpy as jnp
from jax import lax
from jax.experimental import pallas as pl
from jax.experimental.pallas import tpu as pltpu
from jax.experimental.pallas import tpu_sc as plsc

_N = 16384
_EMB = 64
_PAIR = 128
_VBLK = 32768
_Q = _VBLK // 4
_ENT_V = 1000000
_ENT_GRID = -(-_ENT_V // _VBLK)
_ENT_ROWS = _ENT_GRID * _Q
_REL_ROWS = _Q
_NC = 2
_NS = 16
_NW = _NC * _NS
_BPW = _N // _NW
_CHUNK = 128
_L = 16
_SH = _VBLK.bit_length() - 1
_QSH = _SH - 2
_QMASK = _Q - 1


def _pack16(lo, hi):
  lo16 = lax.bitcast_convert_type(lo.astype(jnp.bfloat16), jnp.uint16)
  hi16 = lax.bitcast_convert_type(hi.astype(jnp.bfloat16), jnp.uint16)
  u = lo16.astype(jnp.uint32) | (hi16.astype(jnp.uint32) << 16)
  return lax.bitcast_convert_type(u, jnp.float32)


def _proj_body(xt_ref, w_ref, b_ref, out_ref):
  xt = xt_ref[...]
  w = w_ref[...]
  b = b_ref[...]
  dn = (((0,), (0,)), ((), ()))
  ys = []
  for t in range(4):
    y = lax.dot_general(xt[:, t * _Q:(t + 1) * _Q], w, dn,
                        preferred_element_type=jnp.float32)
    ys.append(y + b)
  out_ref[:, :_EMB] = _pack16(ys[0], ys[1])
  out_ref[:, _EMB:] = _pack16(ys[2], ys[3])


def _project(table, W, b, grid, out_rows):
  tt = table.T
  return pl.pallas_call(
      _proj_body,
      grid=(grid,),
      in_specs=[
          pl.BlockSpec((_EMB, _VBLK), lambda j: (0, j)),
          pl.BlockSpec((_EMB, _EMB), lambda j: (0, 0)),
          pl.BlockSpec((1, _EMB), lambda j: (0, 0)),
      ],
      out_specs=pl.BlockSpec((_Q, _PAIR), lambda j: (j, 0)),
      out_shape=jax.ShapeDtypeStruct((out_rows, _PAIR), jnp.float32),
  )(tt, W, b.reshape(1, _EMB))


def _sc_gather(s, r, o, p4_ent, p4_rel):
  mesh = plsc.VectorSubcoreMesh(
      core_axis_name="c", subcore_axis_name="s",
      num_cores=_NC, num_subcores=_NS)

  @functools.partial(
      pl.kernel,
      out_type=[jax.ShapeDtypeStruct((_N, _PAIR), jnp.float32)] * 3,
      mesh=mesh,
      scratch_types=[
          pltpu.VMEM((_BPW,), jnp.int32),
          pltpu.VMEM((_BPW,), jnp.int32),
          pltpu.VMEM((_BPW,), jnp.int32),
          pltpu.VMEM((_BPW // 2, _PAIR), jnp.float32),
          pltpu.VMEM((_BPW // 2, _PAIR), jnp.float32),
          pltpu.VMEM((_BPW // 2, _PAIR), jnp.float32),
          pltpu.SemaphoreType.DMA,
          pltpu.SemaphoreType.DMA,
      ],
  )
  def k(s_h, r_h, o_h, ent_h, rel_h, xs_h, xr_h, xo_h,
        si_v, ri_v, oi_v, gs_v, gr_v, go_v, gsem, wsem):
    wid = lax.axis_index("s") * _NC + lax.axis_index("c")
    base = wid * _BPW
    pltpu.sync_copy(s_h.at[pl.ds(base, _BPW)], si_v)
    pltpu.sync_copy(r_h.at[pl.ds(base, _BPW)], ri_v)
    pltpu.sync_copy(o_h.at[pl.ds(base, _BPW)], oi_v)
    for iv in (si_v, ri_v, oi_v):
      for g in range(_BPW // _L):
        sl = pl.ds(g * _L, _L)
        v = iv[sl]
        iv[sl] = ((v >> _SH) << _QSH) | (v & _QMASK)
    hr = _BPW // 2
    prev_wb = []
    for h in range(2):
      for c in prev_wb:
        c.wait()
      copies = []
      for j in range(hr // _CHUNK):
        isl = pl.ds(h * hr + j * _CHUNK, _CHUNK)
        bsl = pl.ds(j * _CHUNK, _CHUNK)
        copies.append(
            pltpu.async_copy(ent_h.at[si_v.at[isl]], gs_v.at[bsl], gsem))
        copies.append(
            pltpu.async_copy(rel_h.at[ri_v.at[isl]], gr_v.at[bsl], gsem))
        copies.append(
            pltpu.async_copy(ent_h.at[oi_v.at[isl]], go_v.at[bsl], gsem))
      for c in copies:
        c.wait()
      osl = pl.ds(base + h * hr, hr)
      prev_wb = [pltpu.async_copy(gs_v, xs_h.at[osl], wsem),
                 pltpu.async_copy(gr_v, xr_h.at[osl], wsem),
                 pltpu.async_copy(go_v, xo_h.at[osl], wsem)]
    for c in prev_wb:
      c.wait()

  return k(s, r, o, p4_ent, p4_rel)


_BLK = 2048


def _sel_body(xs_ref, xr_ref, xo_ref, s_ref, r_ref, o_ref, out_ref):
  for col, x_ref, i_ref in ((0, xs_ref, s_ref), (1, xr_ref, r_ref),
                            (2, xo_ref, o_ref)):
    x2 = x_ref[...]
    idx = i_ref[...]
    grp = (idx >> (_QSH + 1)) & 1
    hi = (idx >> _QSH) & 1
    xh = jnp.where(grp > 0, x2[:, _EMB:], x2[:, :_EMB])
    u = lax.bitcast_convert_type(xh, jnp.uint32)
    picked = jnp.where(hi > 0, u & jnp.uint32(0xFFFF0000), u << 16)
    sel = lax.bitcast_convert_type(picked, jnp.float32)
    out_ref[col * _EMB:(col + 1) * _EMB, :] = sel.T


def _select_concat(xs2, xr2, xo2, s, r, o):
  row_spec = pl.BlockSpec((_BLK, _PAIR), lambda i: (i, 0))
  i_spec = pl.BlockSpec((_BLK, 1), lambda i: (i, 0))
  return pl.pallas_call(
      _sel_body,
      grid=(_N // _BLK,),
      in_specs=[row_spec, row_spec, row_spec, i_spec, i_spec, i_spec],
      out_specs=pl.BlockSpec((3 * _EMB, _BLK), lambda i: (0, i)),
      out_shape=jax.ShapeDtypeStruct((3 * _EMB, _N), jnp.float32),
  )(xs2, xr2, xo2, s.reshape(_N, 1), r.reshape(_N, 1), o.reshape(_N, 1)).T


def kernel(s, r, o, entity_table, relation_table, W, b):
  s = s.astype(jnp.int32)
  r = r.astype(jnp.int32)
  o = o.astype(jnp.int32)
  p4_ent = _project(entity_table, W, b, _ENT_GRID, _ENT_ROWS)
  p4_rel = _project(relation_table, W, b, 1, _REL_ROWS)
  xs2, xr2, xo2 = _sc_gather(s, r, o, p4_ent, p4_rel)
  return _select_concat(xs2, xr2, xo2, s, r, o)


# speedup vs baseline: 1.2108x; 1.1556x over previous
import functools

import jax
import jax.numpy as jnp
from jax import lax
from jax.experimental import pallas as pl
from jax.experimental.pallas import tpu as pltpu
from jax.experimental.pallas import tpu_sc as plsc

_N = 16384
_EMB = 64
_PAIR = 128
_VBLK = 32768
_Q = _VBLK // 4
_ENT_V = 1000000
_ENT_GRID = -(-_ENT_V // _VBLK)
_ENT_ROWS = _ENT_GRID * _Q
_REL_ROWS = _Q
_NC = 2
_NS = 16
_NW = _NC * _NS
_BPW = _N // _NW
_CHUNK = 128
_L = 16
_SH = _VBLK.bit_length() - 1
_QSH = _SH - 2
_QMASK = _Q - 1


def _pack16(lo, hi):
  lo16 = lax.bitcast_convert_type(lo.astype(jnp.bfloat16), jnp.uint16)
  hi16 = lax.bitcast_convert_type(hi.astype(jnp.bfloat16), jnp.uint16)
  u = lo16.astype(jnp.uint32) | (hi16.astype(jnp.uint32) << 16)
  return lax.bitcast_convert_type(u, jnp.float32)


def _proj_body(xt_ref, w_ref, b_ref, out_ref):
  xt = xt_ref[...].astype(jnp.bfloat16)
  w = w_ref[...].astype(jnp.bfloat16)
  b = b_ref[...]
  dn = (((0,), (0,)), ((), ()))
  ys = []
  for t in range(4):
    y = lax.dot_general(xt[:, t * _Q:(t + 1) * _Q], w, dn,
                        preferred_element_type=jnp.float32)
    ys.append(y + b)
  out_ref[:, :_EMB] = _pack16(ys[0], ys[1])
  out_ref[:, _EMB:] = _pack16(ys[2], ys[3])


def _project(table, W, b, grid, out_rows):
  tt = table.T
  return pl.pallas_call(
      _proj_body,
      grid=(grid,),
      in_specs=[
          pl.BlockSpec((_EMB, _VBLK), lambda j: (0, j)),
          pl.BlockSpec((_EMB, _EMB), lambda j: (0, 0)),
          pl.BlockSpec((1, _EMB), lambda j: (0, 0)),
      ],
      out_specs=pl.BlockSpec((_Q, _PAIR), lambda j: (j, 0)),
      out_shape=jax.ShapeDtypeStruct((out_rows, _PAIR), jnp.float32),
  )(tt, W, b.reshape(1, _EMB))


def _sc_gather(s, r, o, p4_ent, p4_rel):
  mesh = plsc.VectorSubcoreMesh(
      core_axis_name="c", subcore_axis_name="s",
      num_cores=_NC, num_subcores=_NS)

  @functools.partial(
      pl.kernel,
      out_type=[jax.ShapeDtypeStruct((_N, _PAIR), jnp.float32)] * 3,
      mesh=mesh,
      scratch_types=[
          pltpu.VMEM((_BPW,), jnp.int32),
          pltpu.VMEM((_BPW,), jnp.int32),
          pltpu.VMEM((_BPW,), jnp.int32),
          pltpu.VMEM((_BPW // 2, _PAIR), jnp.float32),
          pltpu.VMEM((_BPW // 2, _PAIR), jnp.float32),
          pltpu.VMEM((_BPW // 2, _PAIR), jnp.float32),
          pltpu.SemaphoreType.DMA,
          pltpu.SemaphoreType.DMA,
      ],
  )
  def k(s_h, r_h, o_h, ent_h, rel_h, xs_h, xr_h, xo_h,
        si_v, ri_v, oi_v, gs_v, gr_v, go_v, gsem, wsem):
    wid = lax.axis_index("s") * _NC + lax.axis_index("c")
    base = wid * _BPW
    pltpu.sync_copy(s_h.at[pl.ds(base, _BPW)], si_v)
    pltpu.sync_copy(r_h.at[pl.ds(base, _BPW)], ri_v)
    pltpu.sync_copy(o_h.at[pl.ds(base, _BPW)], oi_v)
    for iv in (si_v, ri_v, oi_v):
      for g in range(_BPW // _L):
        sl = pl.ds(g * _L, _L)
        v = iv[sl]
        iv[sl] = ((v >> _SH) << _QSH) | (v & _QMASK)
    hr = _BPW // 2
    prev_wb = []
    for h in range(2):
      for c in prev_wb:
        c.wait()
      copies = []
      for j in range(hr // _CHUNK):
        isl = pl.ds(h * hr + j * _CHUNK, _CHUNK)
        bsl = pl.ds(j * _CHUNK, _CHUNK)
        copies.append(
            pltpu.async_copy(ent_h.at[si_v.at[isl]], gs_v.at[bsl], gsem))
        copies.append(
            pltpu.async_copy(rel_h.at[ri_v.at[isl]], gr_v.at[bsl], gsem))
        copies.append(
            pltpu.async_copy(ent_h.at[oi_v.at[isl]], go_v.at[bsl], gsem))
      for c in copies:
        c.wait()
      osl = pl.ds(base + h * hr, hr)
      prev_wb = [pltpu.async_copy(gs_v, xs_h.at[osl], wsem),
                 pltpu.async_copy(gr_v, xr_h.at[osl], wsem),
                 pltpu.async_copy(go_v, xo_h.at[osl], wsem)]
    for c in prev_wb:
      c.wait()

  return k(s, r, o, p4_ent, p4_rel)


_BLK = 2048


def _sel_body(xs_ref, xr_ref, xo_ref, s_ref, r_ref, o_ref, out_ref):
  for col, x_ref, i_ref in ((0, xs_ref, s_ref), (1, xr_ref, r_ref),
                            (2, xo_ref, o_ref)):
    x2 = x_ref[...]
    idx = i_ref[...]
    grp = (idx >> (_QSH + 1)) & 1
    hi = (idx >> _QSH) & 1
    xh = jnp.where(grp > 0, x2[:, _EMB:], x2[:, :_EMB])
    u = lax.bitcast_convert_type(xh, jnp.uint32)
    picked = jnp.where(hi > 0, u & jnp.uint32(0xFFFF0000), u << 16)
    sel = lax.bitcast_convert_type(picked, jnp.float32)
    out_ref[col * _EMB:(col + 1) * _EMB, :] = sel.T


def _select_concat(xs2, xr2, xo2, s, r, o):
  row_spec = pl.BlockSpec((_BLK, _PAIR), lambda i: (i, 0))
  i_spec = pl.BlockSpec((_BLK, 1), lambda i: (i, 0))
  return pl.pallas_call(
      _sel_body,
      grid=(_N // _BLK,),
      in_specs=[row_spec, row_spec, row_spec, i_spec, i_spec, i_spec],
      out_specs=pl.BlockSpec((3 * _EMB, _BLK), lambda i: (0, i)),
      out_shape=jax.ShapeDtypeStruct((3 * _EMB, _N), jnp.float32),
  )(xs2, xr2, xo2, s.reshape(_N, 1), r.reshape(_N, 1), o.reshape(_N, 1)).T


def kernel(s, r, o, entity_table, relation_table, W, b):
  s = s.astype(jnp.int32)
  r = r.astype(jnp.int32)
  o = o.astype(jnp.int32)
  p4_ent = _project(entity_table, W, b, _ENT_GRID, _ENT_ROWS)
  p4_rel = _project(relation_table, W, b, 1, _REL_ROWS)
  xs2, xr2, xo2 = _sc_gather(s, r, o, p4_ent, p4_rel)
  return _select_concat(xs2, xr2, xo2, s, r, o)
